# Initial kernel scaffold; baseline (speedup 1.0000x reference)
import functools

import jax
import jax.numpy as jnp
from jax import lax
from jax.experimental import pallas as pl
from jax.experimental.pallas import tpu as pltpu
from jax.experimental.pallas import tpu_sc as plsc

N = 10000
E = 320000
IN_DIM = 128
HID = 128
EPS = 0.3

NC = 2
NS = 16
NW = NC * NS
CHUNK = 512
NCHUNKS = E // CHUNK
ROWS_PER_TILE = N // NS

_mesh = plsc.VectorSubcoreMesh(core_axis_name="c", subcore_axis_name="s")


def _deg_body(dst2d, degp, ones_v, zbuf, dstL, deg_sp, sem):
    c = lax.axis_index("c")
    s = lax.axis_index("s")
    w = c * NS + s

    def fill(i, _):
        ones_v[i, :] = jnp.full((16,), 1.0, jnp.float32)
        return _
    lax.fori_loop(0, 128, fill, None)

    def zfill(i, _):
        zbuf[i, :] = jnp.zeros((16,), jnp.float32)
        return _
    lax.fori_loop(0, ROWS_PER_TILE, zfill, None)

    pltpu.sync_copy(zbuf, deg_sp.at[pl.ds(s * ROWS_PER_TILE, ROWS_PER_TILE)])
    plsc.subcore_barrier()

    nc_ = jnp.where(w < NCHUNKS % NW, NCHUNKS // NW + 1, NCHUNKS // NW)

    def chunk(j, _):
        crow = (w + NW * j) * (CHUNK // 128)
        pltpu.sync_copy(dst2d.at[pl.ds(crow, CHUNK // 128)], dstL)
        for q in range(CHUNK // 128):
            pltpu.sync_copy(ones_v, deg_sp.at[dstL.at[q]], add=True)
        return _
    lax.fori_loop(0, nc_, chunk, None)

    plsc.subcore_barrier()
    pltpu.sync_copy(deg_sp.at[pl.ds(s * ROWS_PER_TILE, ROWS_PER_TILE)],
                    degp.at[c, pl.ds(s * ROWS_PER_TILE, ROWS_PER_TILE)])


_deg_call = pl.kernel(
    _deg_body,
    out_type=jax.ShapeDtypeStruct((NC, N, 16), jnp.float32),
    mesh=_mesh,
    scratch_types=[
        pltpu.VMEM((128, 16), jnp.float32),
        pltpu.VMEM((ROWS_PER_TILE, 16), jnp.float32),
        pltpu.VMEM((CHUNK // 128, 128), jnp.int32),
        pltpu.VMEM_SHARED((N, 16), jnp.float32),
        pltpu.SemaphoreType.DMA,
    ],
)


def _prep_body(x_ref, W1_ref, b1_ref, gateW_ref, gateb_ref, degp_ref,
               h_ref, abn_ref):
    x = x_ref[...]
    W1 = W1_ref[...]
    h = lax.dot_general(x, W1, (((1,), (1,)), ((), ())),
                        preferred_element_type=jnp.float32)
    h = jnp.maximum(h + b1_ref[...][None, :], 0.0)
    h_ref[...] = h
    gw = gateW_ref[...]
    W2 = jnp.concatenate([gw[:, :HID], gw[:, HID:]], axis=0)
    ab = lax.dot_general(h, W2, (((1,), (1,)), ((), ())),
                         preferred_element_type=jnp.float32)
    ab = ab + jnp.concatenate(
        [jnp.broadcast_to(gateb_ref[...][0:1], (N, 1)),
         jnp.zeros((N, 1), jnp.float32)], axis=1)
    deg = degp_ref[0, :, 0:1] + degp_ref[1, :, 0:1]
    norm = lax.rsqrt(jnp.maximum(deg, 1.0))
    abn_ref[...] = jnp.concatenate([ab, norm, jnp.zeros_like(norm)], axis=1)


_prep_call = pl.pallas_call(
    _prep_body,
    out_shape=[
        jax.ShapeDtypeStruct((N, HID), jnp.float32),
        jax.ShapeDtypeStruct((N, 4), jnp.float32),
    ],
)


def _edge_body(src2d, dst2d, h_hbm, abn_hbm, zp, abn_v, dstL, srcL, rows_v,
               c_v, z_sp, sem):
    c = lax.axis_index("c")
    s = lax.axis_index("s")
    w = c * NS + s

    pltpu.sync_copy(abn_hbm, abn_v)

    def zr(i, _):
        for f in range(HID // 16):
            rows_v[i, pl.ds(f * 16, 16)] = jnp.zeros((16,), jnp.float32)
        return _
    lax.fori_loop(0, CHUNK, zr, None)
    pltpu.sync_copy(rows_v, z_sp.at[pl.ds(s * ROWS_PER_TILE, CHUNK)])
    pltpu.sync_copy(rows_v.at[pl.ds(0, ROWS_PER_TILE - CHUNK)],
                    z_sp.at[pl.ds(s * ROWS_PER_TILE + CHUNK,
                                  ROWS_PER_TILE - CHUNK)])
    plsc.subcore_barrier()

    nc_ = jnp.where(w < NCHUNKS % NW, NCHUNKS // NW + 1, NCHUNKS // NW)
    col0 = jnp.zeros((16,), jnp.int32)
    col1 = jnp.full((16,), 1, jnp.int32)
    col2 = jnp.full((16,), 2, jnp.int32)

    def chunk(j, _):
        crow = (w + NW * j) * (CHUNK // 128)
        pltpu.sync_copy(dst2d.at[pl.ds(crow, CHUNK // 128)], dstL)
        pltpu.sync_copy(src2d.at[pl.ds(crow, CHUNK // 128)], srcL)
        cps = [pltpu.async_copy(h_hbm.at[srcL.at[q]],
                                rows_v.at[pl.ds(q * 128, 128)], sem)
               for q in range(CHUNK // 128)]
        for i in range(CHUNK // 16):
            q, o = divmod(i * 16, 128)
            d16 = dstL[q, pl.ds(o, 16)]
            s16 = srcL[q, pl.ds(o, 16)]
            av = plsc.load_gather(abn_v, [d16, col0])
            bv = plsc.load_gather(abn_v, [s16, col1])
            nv = plsc.load_gather(abn_v, [s16, col2])
            u = av + bv
            ex = jnp.exp(jnp.abs(u) * (-2.0))
            th = (1.0 - ex) / (1.0 + ex)
            t = jnp.where(u < 0.0, -th, th)
            c_v[pl.ds(i * 16, 16)] = t * nv
        for cp in cps:
            cp.wait()
        def se(e, _):
            c16 = plsc.load_gather(c_v, [jnp.broadcast_to(e, (16,))])
            for f in range(HID // 16):
                rows_v[e, pl.ds(f * 16, 16)] = (
                    rows_v[e, pl.ds(f * 16, 16)] * c16)
            return _
        lax.fori_loop(0, CHUNK, se, None)
        for q in range(CHUNK // 128):
            pltpu.sync_copy(rows_v.at[pl.ds(q * 128, 128)],
                            z_sp.at[dstL.at[q]], add=True)
        return _
    lax.fori_loop(0, nc_, chunk, None)

    plsc.subcore_barrier()
    pltpu.sync_copy(z_sp.at[pl.ds(s * ROWS_PER_TILE, ROWS_PER_TILE)],
                    zp.at[c, pl.ds(s * ROWS_PER_TILE, ROWS_PER_TILE)])


_edge_call = pl.kernel(
    _edge_body,
    out_type=jax.ShapeDtypeStruct((NC, N, HID), jnp.float32),
    mesh=_mesh,
    scratch_types=[
        pltpu.VMEM((N, 4), jnp.float32),
        pltpu.VMEM((CHUNK // 128, 128), jnp.int32),
        pltpu.VMEM((CHUNK // 128, 128), jnp.int32),
        pltpu.VMEM((CHUNK, HID), jnp.float32),
        pltpu.VMEM((CHUNK,), jnp.float32),
        pltpu.VMEM_SHARED((N, HID), jnp.float32),
        pltpu.SemaphoreType.DMA,
    ],
)


def _final_body(h_ref, abn_ref, zp_ref, out_ref):
    norm = abn_ref[:, 2:3]
    out_ref[...] = (EPS * h_ref[...]
                    + norm * (zp_ref[0, :, :] + zp_ref[1, :, :]))


_final_call = pl.pallas_call(
    _final_body,
    out_shape=jax.ShapeDtypeStruct((N, HID), jnp.float32),
)


@jax.jit
def kernel(x, edge_index, W1, b1, gateW, gateb):
    src = edge_index[0].astype(jnp.int32).reshape(E // 128, 128)
    dst = edge_index[1].astype(jnp.int32).reshape(E // 128, 128)
    degp = _deg_call(dst)
    h, abn = _prep_call(x, W1, b1, gateW, gateb, degp)
    zp = _edge_call(src, dst, h, abn)
    return _final_call(h, abn, zp)


# speedup vs baseline: 19.0577x; 19.0577x over previous
import jax
import jax.numpy as jnp
from jax import lax
from jax.experimental import pallas as pl
from jax.experimental.pallas import tpu as pltpu
from jax.experimental.pallas import tpu_sc as plsc

N = 10000
E = 320000
IN_DIM = 128
HID = 128
EPS = 0.3

NC = 2
NS = 16
NW = NC * NS
PAIR = 1024
WAVE = 256
NPAIRS = E // PAIR
LEFT_ROW = NPAIRS * 8
ROWS_A = 632
ROWS_B = N - 15 * ROWS_A

_mesh = plsc.VectorSubcoreMesh(core_axis_name="c", subcore_axis_name="s")


def _npairs_of(w):
    return jnp.where(w < NPAIRS % NW, NPAIRS // NW + 1, NPAIRS // NW)


def _deg_body(dst2d, degp, ones_v, zbuf, dstL, deg_sp, sem):
    c = lax.axis_index("c")
    s = lax.axis_index("s")
    w = c * NS + s

    def fill(i, _):
        ones_v[i, :] = jnp.full((16,), 1.0, jnp.float32)
        return _
    lax.fori_loop(0, 128, fill, None)

    def zfill(i, _):
        zbuf[i, :] = jnp.zeros((16,), jnp.float32)
        return _
    lax.fori_loop(0, ROWS_A, zfill, None)

    base = pl.multiple_of(s * ROWS_A, 8)

    @pl.when(s < NS - 1)
    def _():
        pltpu.sync_copy(zbuf, deg_sp.at[pl.ds(base, ROWS_A)])

    @pl.when(s == NS - 1)
    def _():
        pltpu.sync_copy(zbuf.at[pl.ds(0, ROWS_B)],
                        deg_sp.at[pl.ds((NS - 1) * ROWS_A, ROWS_B)])
    plsc.subcore_barrier()

    def scat4(q0):
        for q in range(4):
            pltpu.sync_copy(ones_v, deg_sp.at[dstL.at[q0 + q]], add=True)

    def pair(j, _):
        crow = pl.multiple_of((w + NW * j) * 8, 8)
        pltpu.sync_copy(dst2d.at[pl.ds(crow, 8)], dstL)
        scat4(0)
        scat4(4)
        return _
    lax.fori_loop(0, _npairs_of(w), pair, None)

    @pl.when(w == NW - 1)
    def _():
        pltpu.sync_copy(dst2d.at[pl.ds(LEFT_ROW, 4)], dstL.at[pl.ds(0, 4)])
        scat4(0)

    plsc.subcore_barrier()

    @pl.when(s < NS - 1)
    def _():
        pltpu.sync_copy(deg_sp.at[pl.ds(base, ROWS_A)],
                        degp.at[c, pl.ds(base, ROWS_A)])

    @pl.when(s == NS - 1)
    def _():
        pltpu.sync_copy(deg_sp.at[pl.ds((NS - 1) * ROWS_A, ROWS_B)],
                        degp.at[c, pl.ds((NS - 1) * ROWS_A, ROWS_B)])


_deg_call = pl.kernel(
    _deg_body,
    out_type=jax.ShapeDtypeStruct((NC, N, 16), jnp.float32),
    mesh=_mesh,
    scratch_types=[
        pltpu.VMEM((128, 16), jnp.float32),
        pltpu.VMEM((ROWS_A, 16), jnp.float32),
        pltpu.VMEM((8, 128), jnp.int32),
        pltpu.VMEM_SHARED((N, 16), jnp.float32),
        pltpu.SemaphoreType.DMA,
    ],
    compiler_params=pltpu.CompilerParams(needs_layout_passes=False,
                                         use_tc_tiling_on_sc=False),
)


def _prep_body(x_ref, W1_ref, b1_ref, gateW_ref, gateb_ref, degp_ref,
               h_ref, abn_ref):
    x = x_ref[...]
    W1 = W1_ref[...]
    h = lax.dot_general(x, W1, (((1,), (1,)), ((), ())),
                        preferred_element_type=jnp.float32)
    h = jnp.maximum(h + b1_ref[...][None, :], 0.0)
    h_ref[...] = h
    gw = gateW_ref[...]
    W2 = jnp.concatenate([gw[:, :HID], gw[:, HID:]], axis=0)
    ab = lax.dot_general(h, W2, (((1,), (1,)), ((), ())),
                         preferred_element_type=jnp.float32)
    gb = jnp.broadcast_to(gateb_ref[...][0:1], (N, 1))
    ab = ab + jnp.concatenate([gb, jnp.zeros((N, 1), jnp.float32)], axis=1)
    deg = degp_ref[0, :, 0:1] + degp_ref[1, :, 0:1]
    norm = lax.rsqrt(jnp.maximum(deg, 1.0))
    abn_ref[...] = jnp.concatenate(
        [ab, norm, jnp.zeros((N, 13), jnp.float32)], axis=1)


_prep_call = pl.pallas_call(
    _prep_body,
    out_shape=[
        jax.ShapeDtypeStruct((N, HID), jnp.float32),
        jax.ShapeDtypeStruct((N, 16), jnp.float32),
    ],
)


def _edge_body(src2d, dst2d, h_hbm, abn_hbm, zp, dstL, srcL, rows_v,
               ad_buf, as_buf, c_v, z_sp, sem):
    c = lax.axis_index("c")
    s = lax.axis_index("s")
    w = c * NS + s

    def zr(i, _):
        for f in range(HID // 16):
            rows_v[i, pl.ds(f * 16, 16)] = jnp.zeros((16,), jnp.float32)
        return _
    lax.fori_loop(0, WAVE, zr, None)
    base = pl.multiple_of(s * ROWS_A, 8)

    @pl.when(s < NS - 1)
    def _():
        pltpu.sync_copy(rows_v, z_sp.at[pl.ds(base, WAVE)])
        pltpu.sync_copy(rows_v, z_sp.at[pl.ds(base + WAVE, WAVE)])
        pltpu.sync_copy(rows_v.at[pl.ds(0, ROWS_A - 2 * WAVE)],
                        z_sp.at[pl.ds(base + 2 * WAVE, ROWS_A - 2 * WAVE)])

    @pl.when(s == NS - 1)
    def _():
        b15 = (NS - 1) * ROWS_A
        pltpu.sync_copy(rows_v, z_sp.at[pl.ds(b15, WAVE)])
        pltpu.sync_copy(rows_v, z_sp.at[pl.ds(b15 + WAVE, WAVE)])
        pltpu.sync_copy(rows_v.at[pl.ds(0, ROWS_B - 2 * WAVE)],
                        z_sp.at[pl.ds(b15 + 2 * WAVE, ROWS_B - 2 * WAVE)])
    plsc.subcore_barrier()

    iota = lax.iota(jnp.int32, 16)
    col0 = jnp.zeros((16,), jnp.int32)
    col1 = jnp.full((16,), 1, jnp.int32)
    col2 = jnp.full((16,), 2, jnp.int32)

    def do_wave(v):
        cps = []
        for q in range(2):
            r = 2 * v + q
            cps.append(pltpu.async_copy(h_hbm.at[srcL.at[r]],
                                        rows_v.at[pl.ds(q * 128, 128)], sem))
            cps.append(pltpu.async_copy(abn_hbm.at[dstL.at[r]],
                                        ad_buf.at[pl.ds(q * 128, 128)], sem))
            cps.append(pltpu.async_copy(abn_hbm.at[srcL.at[r]],
                                        as_buf.at[pl.ds(q * 128, 128)], sem))
        for cp in cps:
            cp.wait()
        for i in range(WAVE // 16):
            ridx = iota + (i * 16)
            av = plsc.load_gather(ad_buf, [ridx, col0])
            bv = plsc.load_gather(as_buf, [ridx, col1])
            nv = plsc.load_gather(as_buf, [ridx, col2])
            u = av + bv
            ex = jnp.exp(jnp.abs(u) * (-2.0))
            th = (1.0 - ex) / (1.0 + ex)
            t = jnp.where(u < 0.0, -th, th)
            c_v[pl.ds(i * 16, 16)] = t * nv

        def se(e, _):
            c16 = plsc.load_gather(c_v, [jnp.broadcast_to(e, (16,))])
            for f in range(HID // 16):
                rows_v[e, pl.ds(f * 16, 16)] = (
                    rows_v[e, pl.ds(f * 16, 16)] * c16)
            return _
        lax.fori_loop(0, WAVE, se, None)

        for q in range(2):
            pltpu.sync_copy(rows_v.at[pl.ds(q * 128, 128)],
                            z_sp.at[dstL.at[2 * v + q]], add=True)

    def pair(j, _):
        crow = pl.multiple_of((w + NW * j) * 8, 8)
        pltpu.sync_copy(dst2d.at[pl.ds(crow, 8)], dstL)
        pltpu.sync_copy(src2d.at[pl.ds(crow, 8)], srcL)
        for v in range(4):
            do_wave(v)
        return _
    lax.fori_loop(0, _npairs_of(w), pair, None)

    @pl.when(w == NW - 1)
    def _():
        pltpu.sync_copy(dst2d.at[pl.ds(LEFT_ROW, 4)], dstL.at[pl.ds(0, 4)])
        pltpu.sync_copy(src2d.at[pl.ds(LEFT_ROW, 4)], srcL.at[pl.ds(0, 4)])
        do_wave(0)
        do_wave(1)

    plsc.subcore_barrier()

    @pl.when(s < NS - 1)
    def _():
        pltpu.sync_copy(z_sp.at[pl.ds(base, ROWS_A)],
                        zp.at[c, pl.ds(base, ROWS_A)])

    @pl.when(s == NS - 1)
    def _():
        pltpu.sync_copy(z_sp.at[pl.ds((NS - 1) * ROWS_A, ROWS_B)],
                        zp.at[c, pl.ds((NS - 1) * ROWS_A, ROWS_B)])


_edge_call = pl.kernel(
    _edge_body,
    out_type=jax.ShapeDtypeStruct((NC, N, HID), jnp.float32),
    mesh=_mesh,
    scratch_types=[
        pltpu.VMEM((8, 128), jnp.int32),
        pltpu.VMEM((8, 128), jnp.int32),
        pltpu.VMEM((WAVE, HID), jnp.float32),
        pltpu.VMEM((WAVE, 16), jnp.float32),
        pltpu.VMEM((WAVE, 16), jnp.float32),
        pltpu.VMEM((WAVE,), jnp.float32),
        pltpu.VMEM_SHARED((N, HID), jnp.float32),
        pltpu.SemaphoreType.DMA,
    ],
    compiler_params=pltpu.CompilerParams(needs_layout_passes=False,
                                         use_tc_tiling_on_sc=False),
)


def _final_body(h_ref, abn_ref, zp_ref, out_ref):
    norm = abn_ref[:, 2:3]
    out_ref[...] = (EPS * h_ref[...]
                    + norm * (zp_ref[0, :, :] + zp_ref[1, :, :]))


_final_call = pl.pallas_call(
    _final_body,
    out_shape=jax.ShapeDtypeStruct((N, HID), jnp.float32),
)


@jax.jit
def kernel(x, edge_index, W1, b1, gateW, gateb):
    src = edge_index[0].astype(jnp.int32).reshape(E // 128, 128)
    dst = edge_index[1].astype(jnp.int32).reshape(E // 128, 128)
    degp = _deg_call(dst)
    h, abn = _prep_call(x, W1, b1, gateW, gateb, degp)
    zp = _edge_call(src, dst, h, abn)
    return _final_call(h, abn, zp)


# speedup vs baseline: 24.6114x; 1.2914x over previous
import jax
import jax.numpy as jnp
from jax import lax
from jax.experimental import pallas as pl
from jax.experimental.pallas import tpu as pltpu
from jax.experimental.pallas import tpu_sc as plsc

N = 10000
E = 320000
IN_DIM = 128
HID = 128
EPS = 0.3

NC = 2
NS = 16
NW = NC * NS
PAIR = 1024
WAVE = 128
NPAIRS = E // PAIR
LEFT_ROW = NPAIRS * 8
ROWS_A = 632
ROWS_B = N - 15 * ROWS_A

_mesh = plsc.VectorSubcoreMesh(core_axis_name="c", subcore_axis_name="s")


def _npairs_of(w):
    return jnp.where(w < NPAIRS % NW, NPAIRS // NW + 1, NPAIRS // NW)


def _deg_body(dst2d, degp, ones_v, zbuf, dstL, deg_sp, sem):
    c = lax.axis_index("c")
    s = lax.axis_index("s")
    w = c * NS + s

    def fill(i, _):
        ones_v[i, :] = jnp.full((16,), 1.0, jnp.float32)
        return _
    lax.fori_loop(0, 128, fill, None)

    def zfill(i, _):
        zbuf[i, :] = jnp.zeros((16,), jnp.float32)
        return _
    lax.fori_loop(0, ROWS_A, zfill, None)

    base = pl.multiple_of(s * ROWS_A, 8)

    @pl.when(s < NS - 1)
    def _():
        pltpu.sync_copy(zbuf, deg_sp.at[pl.ds(base, ROWS_A)])

    @pl.when(s == NS - 1)
    def _():
        pltpu.sync_copy(zbuf.at[pl.ds(0, ROWS_B)],
                        deg_sp.at[pl.ds((NS - 1) * ROWS_A, ROWS_B)])
    plsc.subcore_barrier()

    def scat4(q0):
        for q in range(4):
            pltpu.sync_copy(ones_v, deg_sp.at[dstL.at[q0 + q]], add=True)

    def pair(j, _):
        crow = pl.multiple_of((w + NW * j) * 8, 8)
        pltpu.sync_copy(dst2d.at[pl.ds(crow, 8)], dstL)
        scat4(0)
        scat4(4)
        return _
    lax.fori_loop(0, _npairs_of(w), pair, None)

    @pl.when(w == NW - 1)
    def _():
        pltpu.sync_copy(dst2d.at[pl.ds(LEFT_ROW, 4)], dstL.at[pl.ds(0, 4)])
        scat4(0)

    plsc.subcore_barrier()

    @pl.when(s < NS - 1)
    def _():
        pltpu.sync_copy(deg_sp.at[pl.ds(base, ROWS_A)],
                        degp.at[c, pl.ds(base, ROWS_A)])

    @pl.when(s == NS - 1)
    def _():
        pltpu.sync_copy(deg_sp.at[pl.ds((NS - 1) * ROWS_A, ROWS_B)],
                        degp.at[c, pl.ds((NS - 1) * ROWS_A, ROWS_B)])


_deg_call = pl.kernel(
    _deg_body,
    out_type=jax.ShapeDtypeStruct((NC, N, 16), jnp.float32),
    mesh=_mesh,
    scratch_types=[
        pltpu.VMEM((128, 16), jnp.float32),
        pltpu.VMEM((ROWS_A, 16), jnp.float32),
        pltpu.VMEM((8, 128), jnp.int32),
        pltpu.VMEM_SHARED((N, 16), jnp.float32),
        pltpu.SemaphoreType.DMA,
    ],
    compiler_params=pltpu.CompilerParams(needs_layout_passes=False,
                                         use_tc_tiling_on_sc=False),
)


def _prep_body(x_ref, W1_ref, b1_ref, gateW_ref, gateb_ref, degp_ref,
               h_ref, abn_ref):
    x = x_ref[...]
    W1 = W1_ref[...]
    h = lax.dot_general(x, W1, (((1,), (1,)), ((), ())),
                        preferred_element_type=jnp.float32)
    h = jnp.maximum(h + b1_ref[...][None, :], 0.0)
    h_ref[...] = h
    gw = gateW_ref[...]
    W2 = jnp.concatenate([gw[:, :HID], gw[:, HID:]], axis=0)
    ab = lax.dot_general(h, W2, (((1,), (1,)), ((), ())),
                         preferred_element_type=jnp.float32)
    gb = jnp.broadcast_to(gateb_ref[...][0:1], (N, 1))
    ab = ab + jnp.concatenate([gb, jnp.zeros((N, 1), jnp.float32)], axis=1)
    deg = degp_ref[0, :, 0:1] + degp_ref[1, :, 0:1]
    norm = lax.rsqrt(jnp.maximum(deg, 1.0))
    abn_ref[...] = jnp.concatenate(
        [ab, norm, jnp.zeros((N, 13), jnp.float32)], axis=1)


_prep_call = pl.pallas_call(
    _prep_body,
    out_shape=[
        jax.ShapeDtypeStruct((N, HID), jnp.float32),
        jax.ShapeDtypeStruct((N, 16), jnp.float32),
    ],
)


def _edge_body(src2d, dst2d, h_hbm, abn_hbm, zp, dstL, srcL, rows_v,
               ad_buf, as_buf, c_v, z_sp, sem, sem_s):
    c = lax.axis_index("c")
    s = lax.axis_index("s")
    w = c * NS + s

    def zr(i, _):
        for f in range(HID // 16):
            rows_v[i, pl.ds(f * 16, 16)] = jnp.zeros((16,), jnp.float32)
        return _
    lax.fori_loop(0, 2 * WAVE, zr, None)
    base = pl.multiple_of(s * ROWS_A, 8)

    @pl.when(s < NS - 1)
    def _():
        pltpu.sync_copy(rows_v, z_sp.at[pl.ds(base, 2 * WAVE)])
        pltpu.sync_copy(rows_v.at[pl.ds(0, ROWS_A - 2 * WAVE)],
                        z_sp.at[pl.ds(base + 2 * WAVE, ROWS_A - 2 * WAVE)])

    @pl.when(s == NS - 1)
    def _():
        b15 = (NS - 1) * ROWS_A
        pltpu.sync_copy(rows_v, z_sp.at[pl.ds(b15, 2 * WAVE)])
        pltpu.sync_copy(rows_v.at[pl.ds(0, ROWS_B - 2 * WAVE)],
                        z_sp.at[pl.ds(b15 + 2 * WAVE, ROWS_B - 2 * WAVE)])
    plsc.subcore_barrier()

    iota = lax.iota(jnp.int32, 16)
    col0 = jnp.zeros((16,), jnp.int32)
    col1 = jnp.full((16,), 1, jnp.int32)
    col2 = jnp.full((16,), 2, jnp.int32)

    def fire(v):
        p = (v % 2) * 128
        return [pltpu.async_copy(h_hbm.at[srcL.at[v]],
                                 rows_v.at[pl.ds(p, 128)], sem),
                pltpu.async_copy(abn_hbm.at[dstL.at[v]],
                                 ad_buf.at[pl.ds(p, 128)], sem),
                pltpu.async_copy(abn_hbm.at[srcL.at[v]],
                                 as_buf.at[pl.ds(p, 128)], sem)]

    def compute(v):
        p = (v % 2) * 128
        for i in range(128 // 16):
            ridx = iota + (p + i * 16)
            av = plsc.load_gather(ad_buf, [ridx, col0])
            bv = plsc.load_gather(as_buf, [ridx, col1])
            nv = plsc.load_gather(as_buf, [ridx, col2])
            u = av + bv
            ex = jnp.exp(jnp.abs(u) * (-2.0))
            th = (1.0 - ex) / (1.0 + ex)
            t = jnp.where(u < 0.0, -th, th)
            c_v[pl.ds(p + i * 16, 16)] = t * nv

        def se(e, _):
            c16 = plsc.load_gather(c_v, [jnp.broadcast_to(e, (16,))])
            for f in range(HID // 16):
                rows_v[e, pl.ds(f * 16, 16)] = (
                    rows_v[e, pl.ds(f * 16, 16)] * c16)
            return _
        lax.fori_loop(p, p + 128, se, None)
        return pltpu.async_copy(rows_v.at[pl.ds(p, 128)],
                                z_sp.at[dstL.at[v]], sem_s, add=True)

    def pair(j, _):
        crow = pl.multiple_of((w + NW * j) * 8, 8)
        pltpu.sync_copy(dst2d.at[pl.ds(crow, 8)], dstL)
        pltpu.sync_copy(src2d.at[pl.ds(crow, 8)], srcL)
        scats = [None, None]
        cps = fire(0)
        for v in range(8):
            if v < 7:
                if scats[(v + 1) % 2] is not None:
                    scats[(v + 1) % 2].wait()
                nxt = fire(v + 1)
            for cp in cps:
                cp.wait()
            scats[v % 2] = compute(v)
            if v < 7:
                cps = nxt
        scats[0].wait()
        scats[1].wait()
        return _
    lax.fori_loop(0, _npairs_of(w), pair, None)

    @pl.when(w == NW - 1)
    def _():
        pltpu.sync_copy(dst2d.at[pl.ds(LEFT_ROW, 4)], dstL.at[pl.ds(0, 4)])
        pltpu.sync_copy(src2d.at[pl.ds(LEFT_ROW, 4)], srcL.at[pl.ds(0, 4)])
        for v in range(4):
            for cp in fire(v):
                cp.wait()
            compute(v).wait()

    plsc.subcore_barrier()

    @pl.when(s < NS - 1)
    def _():
        pltpu.sync_copy(z_sp.at[pl.ds(base, ROWS_A)],
                        zp.at[c, pl.ds(base, ROWS_A)])

    @pl.when(s == NS - 1)
    def _():
        pltpu.sync_copy(z_sp.at[pl.ds((NS - 1) * ROWS_A, ROWS_B)],
                        zp.at[c, pl.ds((NS - 1) * ROWS_A, ROWS_B)])


_edge_call = pl.kernel(
    _edge_body,
    out_type=jax.ShapeDtypeStruct((NC, N, HID), jnp.float32),
    mesh=_mesh,
    scratch_types=[
        pltpu.VMEM((8, 128), jnp.int32),
        pltpu.VMEM((8, 128), jnp.int32),
        pltpu.VMEM((2 * WAVE, HID), jnp.float32),
        pltpu.VMEM((2 * WAVE, 16), jnp.float32),
        pltpu.VMEM((2 * WAVE, 16), jnp.float32),
        pltpu.VMEM((2 * WAVE,), jnp.float32),
        pltpu.VMEM_SHARED((N, HID), jnp.float32),
        pltpu.SemaphoreType.DMA,
        pltpu.SemaphoreType.DMA,
    ],
    compiler_params=pltpu.CompilerParams(needs_layout_passes=False,
                                         use_tc_tiling_on_sc=False),
)


def _final_body(h_ref, abn_ref, zp_ref, out_ref):
    norm = abn_ref[:, 2:3]
    out_ref[...] = (EPS * h_ref[...]
                    + norm * (zp_ref[0, :, :] + zp_ref[1, :, :]))


_final_call = pl.pallas_call(
    _final_body,
    out_shape=jax.ShapeDtypeStruct((N, HID), jnp.float32),
)


@jax.jit
def kernel(x, edge_index, W1, b1, gateW, gateb):
    src = edge_index[0].astype(jnp.int32).reshape(E // 128, 128)
    dst = edge_index[1].astype(jnp.int32).reshape(E // 128, 128)
    degp = _deg_call(dst)
    h, abn = _prep_call(x, W1, b1, gateW, gateb, degp)
    zp = _edge_call(src, dst, h, abn)
    return _final_call(h, abn, zp)
